# baseline (speedup 1.0000x reference)
import jax
import jax.numpy as jnp
from jax import lax
from jax.experimental import pallas as pl
from jax.experimental.pallas import tpu as pltpu
from jax.experimental.pallas import tpu_sc as plsc

HIDDEN = 1024
EXPERTS = 8
FFN = 4096
TOKENS = 4096
LBW = 0.01

TM = 512
NBLK = TOKENS // TM
NSTEP = NBLK + EXPERTS - 1
FK = 2048
NF = FFN // FK
GW = 128

NC = 2
NS = 16
NW = NC * NS
CHUNK = TOKENS // NW
NGRP = CHUNK // 16

_SC_PARAMS = pltpu.CompilerParams(needs_layout_passes=False)


def _router_body(x_ref, rw_ref, gate_ref, idx_ref, psum_ref):
    i = pl.program_id(0)
    logits = jnp.dot(x_ref[...], rw_ref[...], preferred_element_type=jnp.float32)
    m = jnp.max(logits, axis=1, keepdims=True)
    ex = jnp.exp(logits - m)
    s = jnp.sum(ex, axis=1, keepdims=True)
    probs = ex / s
    gate = jnp.max(probs, axis=1, keepdims=True)
    ids = jax.lax.broadcasted_iota(jnp.int32, probs.shape, 1)
    idx = jnp.min(jnp.where(probs == gate, ids, EXPERTS), axis=1, keepdims=True)
    gate_ref[...] = jnp.broadcast_to(gate, gate_ref.shape)
    idx_ref[...] = jnp.broadcast_to(idx, idx_ref.shape)
    ppsum = jnp.sum(probs, axis=0, keepdims=True)

    @pl.when(i == 0)
    def _():
        psum_ref[...] = ppsum

    @pl.when(i > 0)
    def _():
        psum_ref[...] += ppsum


def _dispatch_body(idx_hbm, gate_hbm, psum_hbm, x_hbm,
                   pos_hbm, xs_hbm, gs_hbm, tb_hbm, be_hbm, off_hbm, aux_hbm,
                   idx_v, gate_v, pos_v, pos8_v, off_s, tb_v, be_v,
                   psum_v, aux_v, xrow_v, grow_v, xsem, gsem):
    wid = lax.axis_index("s") * NC + lax.axis_index("c")
    lane = lax.iota(jnp.int32, 16)
    zeros16 = jnp.zeros((16,), jnp.int32)
    one16 = jnp.ones((16,), jnp.int32)

    def mask_i32(m):
        return jnp.where(m, one16, zeros16)

    def full16(val):
        return jnp.full((16,), val, jnp.int32)

    pltpu.sync_copy(idx_hbm, idx_v)
    pltpu.sync_copy(gate_hbm.at[pl.ds(wid * CHUNK, CHUNK)], gate_v)

    def count_body(g, acc):
        v = idx_v[pl.ds(g * 16, 16)]
        return tuple(acc[e] + mask_i32(v == full16(e)) for e in range(EXPERTS))

    zacc = tuple(zeros16 for _ in range(EXPERTS))
    pre_acc = lax.fori_loop(0, wid * NGRP, count_body, zacc)
    tot_acc = lax.fori_loop(wid * NGRP, TOKENS // 16, count_body, pre_acc)
    pre = [jnp.sum(pre_acc[e]) for e in range(EXPERTS)]
    tot = [jnp.sum(tot_acc[e]) for e in range(EXPERTS)]

    tot_vec = zeros16
    for e in range(EXPERTS):
        tot_vec = tot_vec + jnp.where(lane == full16(e), zeros16 + tot[e],
                                      zeros16)
    off_excl = plsc.cumsum(tot_vec) - tot_vec
    off_s[...] = off_excl

    bases = [off_excl[e] + pre[e] for e in range(EXPERTS)]
    for g in range(NGRP):
        v = idx_v[pl.ds((wid * NGRP + g) * 16, 16)]
        p_vec = zeros16
        for e in range(EXPERTS):
            m = mask_i32(v == full16(e))
            incl = plsc.cumsum(m)
            p_vec = p_vec + m * (bases[e] + incl - 1)
            bases[e] = bases[e] + jnp.sum(m)
        pos_v[pl.ds(g * 16, 16)] = p_vec
        pos8_v[g] = p_vec
    pltpu.sync_copy(pos_v, pos_hbm.at[pl.ds(wid * CHUNK, CHUNK)])

    fz16 = jnp.zeros((16,), jnp.float32)
    for bb in range(2):
        for r in range(16):
            for c in range(GW // 16):
                grow_v[bb, r, pl.ds(c * 16, 16)] = fz16

    xdescs = [None] * NGRP
    gdescs = [None] * NGRP
    for g in range(NGRP):
        if g >= 2:
            xdescs[g - 2].wait()
            gdescs[g - 2].wait()
        pltpu.sync_copy(x_hbm.at[pl.ds(wid * CHUNK + g * 16, 16)],
                        xrow_v.at[g % 2])
        xdescs[g] = pltpu.make_async_copy(xrow_v.at[g % 2],
                                          xs_hbm.at[pos8_v.at[g]], xsem)
        xdescs[g].start()
        gvec = gate_v[pl.ds(g * 16, 16)]
        plsc.store_scatter(grow_v.at[g % 2], [lane, zeros16], gvec)
        gdescs[g] = pltpu.make_async_copy(grow_v.at[g % 2],
                                          gs_hbm.at[pos8_v.at[g]], gsem)
        gdescs[g].start()
    for g in range(NGRP - 2, NGRP):
        xdescs[g].wait()
        gdescs[g].wait()

    @pl.when(wid == 0)
    def _():
        for c in range(2):
            tb_v[pl.ds(c * 16, 16)] = jnp.full((16,), NBLK - 1, jnp.int32)
            be_v[pl.ds(c * 16, 16)] = jnp.full((16,), EXPERTS, jnp.int32)
        running = jnp.int32(0)
        for c in range(NBLK * EXPERTS // 16):
            q = c * 16 + lane
            tb_q = q // EXPERTS
            e_q = q % EXPERTS
            lo = plsc.load_gather(off_s, [e_q])
            hi = plsc.load_gather(off_s, [e_q + 1])
            valid = (lo < (tb_q + 1) * TM) & (hi > tb_q * TM) & (hi > lo)
            mi = mask_i32(valid)
            pos = running + plsc.cumsum(mi) - 1
            plsc.store_scatter(tb_v, [pos], tb_q, mask=valid)
            plsc.store_scatter(be_v, [pos], e_q, mask=valid)
            running = running + jnp.sum(mi)
        pltpu.sync_copy(tb_v, tb_hbm)
        pltpu.sync_copy(be_v, be_hbm)
        pltpu.sync_copy(off_s, off_hbm)
        pltpu.sync_copy(psum_hbm, psum_v)
        prod = psum_v[...] * tot_vec.astype(jnp.float32)
        aux = jnp.sum(prod) * (LBW * EXPERTS / (float(TOKENS) * float(TOKENS)))
        aux_v[...] = jnp.zeros((16,), jnp.float32) + aux
        pltpu.sync_copy(aux_v, aux_hbm)


def _expert_body(tb_ref, be_ref, off_ref, x_ref, gs_ref, w1_ref, b1_ref,
                 w2_ref, b2_ref, out_ref):
    s = pl.program_id(0)
    f = pl.program_id(1)
    be = be_ref[s]
    tb = tb_ref[s]
    lo = off_ref[be]
    hi = off_ref[be + 1]
    rows = jax.lax.broadcasted_iota(jnp.int32, (TM, 1), 0) + tb * TM
    mask = (rows >= lo) & (rows < hi)
    sel = (jax.lax.broadcasted_iota(jnp.int32, (GW, 1), 0) == 0)
    gcol = jnp.dot(gs_ref[...], sel.astype(jnp.float32),
                   precision=jax.lax.Precision.HIGHEST,
                   preferred_element_type=jnp.float32)
    h = jnp.dot(x_ref[...].astype(jnp.bfloat16), w1_ref[0].astype(jnp.bfloat16),
                preferred_element_type=jnp.float32)
    h = jax.nn.gelu(h + b1_ref[0])
    contrib = jnp.dot(h.astype(jnp.bfloat16), w2_ref[0].astype(jnp.bfloat16),
                      preferred_element_type=jnp.float32)
    bias_on = jnp.where(f == 0, 1.0, 0.0).astype(jnp.float32)
    contrib = contrib + bias_on * b2_ref[0]
    delta = jnp.where(mask, gcol * contrib, 0.0)
    prev_tb = tb_ref[jnp.maximum(s - 1, 0)]
    first = (f == 0) & ((s == 0) | (tb != prev_tb))

    @pl.when(first)
    def _():
        out_ref[...] = delta

    @pl.when(jnp.logical_not(first))
    def _():
        out_ref[...] += delta


def _combine_body(pos_hbm, outs_hbm, out_hbm, pos_v, pos8_v, row_v, sem):
    wid = lax.axis_index("s") * NC + lax.axis_index("c")
    pltpu.sync_copy(pos_hbm.at[pl.ds(wid * CHUNK, CHUNK)], pos_v)
    for g in range(NGRP):
        pos8_v[g] = pos_v[pl.ds(g * 16, 16)]
    descs = [None] * NGRP
    descs[0] = pltpu.make_async_copy(outs_hbm.at[pos8_v.at[0]],
                                     row_v.at[0], sem)
    descs[0].start()
    for g in range(NGRP):
        descs[g].wait()
        if g + 1 < NGRP:
            descs[g + 1] = pltpu.make_async_copy(
                outs_hbm.at[pos8_v.at[g + 1]], row_v.at[(g + 1) % 2], sem)
            descs[g + 1].start()
        pltpu.sync_copy(row_v.at[g % 2],
                        out_hbm.at[pl.ds(wid * CHUNK + g * 16, 16)])


def _sc_mesh():
    return plsc.VectorSubcoreMesh(core_axis_name="c", subcore_axis_name="s")


def kernel(x, router_w, fc1_w, fc1_b, fc2_w, fc2_b):
    b, s_, h_ = x.shape
    x_flat = x.reshape(-1, h_)
    mesh = _sc_mesh()

    rb = TOKENS // 8
    gate2d, idx2d, psum = pl.pallas_call(
        _router_body,
        grid=(TOKENS // rb,),
        in_specs=[
            pl.BlockSpec((rb, HIDDEN), lambda i: (i, 0)),
            pl.BlockSpec((HIDDEN, EXPERTS), lambda i: (0, 0)),
        ],
        out_specs=[
            pl.BlockSpec((rb, EXPERTS), lambda i: (i, 0)),
            pl.BlockSpec((rb, EXPERTS), lambda i: (i, 0)),
            pl.BlockSpec((1, EXPERTS), lambda i: (0, 0)),
        ],
        out_shape=[
            jax.ShapeDtypeStruct((TOKENS, EXPERTS), jnp.float32),
            jax.ShapeDtypeStruct((TOKENS, EXPERTS), jnp.int32),
            jax.ShapeDtypeStruct((1, EXPERTS), jnp.float32),
        ],
    )(x_flat, router_w)

    idx = idx2d[:, 0]
    gate = gate2d[:, 0]
    psum16 = jnp.pad(psum.reshape(EXPERTS), (0, 8))

    dispatch = pl.kernel(
        _dispatch_body,
        out_type=[
            jax.ShapeDtypeStruct((TOKENS,), jnp.int32),
            jax.ShapeDtypeStruct((TOKENS, HIDDEN), jnp.float32),
            jax.ShapeDtypeStruct((TOKENS, GW), jnp.float32),
            jax.ShapeDtypeStruct((32,), jnp.int32),
            jax.ShapeDtypeStruct((32,), jnp.int32),
            jax.ShapeDtypeStruct((16,), jnp.int32),
            jax.ShapeDtypeStruct((16,), jnp.float32),
        ],
        mesh=mesh,
        compiler_params=_SC_PARAMS,
        scratch_types=[
            pltpu.VMEM((TOKENS,), jnp.int32),
            pltpu.VMEM((CHUNK,), jnp.float32),
            pltpu.VMEM((CHUNK,), jnp.int32),
            pltpu.VMEM((NGRP, 16), jnp.int32),
            pltpu.VMEM((16,), jnp.int32),
            pltpu.VMEM((32,), jnp.int32),
            pltpu.VMEM((32,), jnp.int32),
            pltpu.VMEM((16,), jnp.float32),
            pltpu.VMEM((16,), jnp.float32),
            pltpu.VMEM((2, 16, HIDDEN), jnp.float32),
            pltpu.VMEM((2, 16, GW), jnp.float32),
            pltpu.SemaphoreType.DMA,
            pltpu.SemaphoreType.DMA,
        ],
    )
    posv, xs, gs, tbv, bev, offv, auxv = dispatch(idx, gate, psum16, x_flat)

    grid_spec = pltpu.PrefetchScalarGridSpec(
        num_scalar_prefetch=3,
        grid=(NSTEP, NF),
        in_specs=[
            pl.BlockSpec((TM, HIDDEN), lambda st, f, tb, be, off: (tb[st], 0)),
            pl.BlockSpec((TM, GW), lambda st, f, tb, be, off: (tb[st], 0)),
            pl.BlockSpec((1, HIDDEN, FK),
                         lambda st, f, tb, be, off: (jnp.minimum(be[st], EXPERTS - 1), 0, f)),
            pl.BlockSpec((1, 1, FK),
                         lambda st, f, tb, be, off: (jnp.minimum(be[st], EXPERTS - 1), 0, f)),
            pl.BlockSpec((1, FK, HIDDEN),
                         lambda st, f, tb, be, off: (jnp.minimum(be[st], EXPERTS - 1), f, 0)),
            pl.BlockSpec((1, 1, HIDDEN),
                         lambda st, f, tb, be, off: (jnp.minimum(be[st], EXPERTS - 1), 0, 0)),
        ],
        out_specs=pl.BlockSpec((TM, HIDDEN), lambda st, f, tb, be, off: (tb[st], 0)),
    )
    outs = pl.pallas_call(
        _expert_body,
        grid_spec=grid_spec,
        out_shape=jax.ShapeDtypeStruct((TOKENS, HIDDEN), jnp.float32),
    )(tbv, bev, offv, xs, gs, fc1_w, fc1_b.reshape(EXPERTS, 1, FFN),
      fc2_w, fc2_b.reshape(EXPERTS, 1, HIDDEN))

    combine = pl.kernel(
        _combine_body,
        out_type=[jax.ShapeDtypeStruct((TOKENS, HIDDEN), jnp.float32)],
        mesh=mesh,
        compiler_params=_SC_PARAMS,
        scratch_types=[
            pltpu.VMEM((CHUNK,), jnp.int32),
            pltpu.VMEM((NGRP, 16), jnp.int32),
            pltpu.VMEM((2, 16, HIDDEN), jnp.float32),
            pltpu.SemaphoreType.DMA,
        ],
    )
    (out,) = combine(posv, outs)

    return out.reshape(b, s_, h_), auxv[0].reshape(())


# speedup vs baseline: 1.0110x; 1.0110x over previous
import jax
import jax.numpy as jnp
from jax import lax
from jax.experimental import pallas as pl
from jax.experimental.pallas import tpu as pltpu
from jax.experimental.pallas import tpu_sc as plsc

HIDDEN = 1024
EXPERTS = 8
FFN = 4096
TOKENS = 4096
LBW = 0.01

TM = 512
NBLKP = TOKENS // TM + EXPERTS - 1
TPAD = NBLKP * TM
FK = 2048
NF = FFN // FK
GW = 128

NC = 2
NS = 16
NW = NC * NS
CHUNK = TOKENS // NW
NGRP = CHUNK // 16

_SC_PARAMS = pltpu.CompilerParams(needs_layout_passes=False)


def _router_body(x_ref, rw_ref, gate_ref, idx_ref, psum_ref):
    i = pl.program_id(0)
    logits = jnp.dot(x_ref[...], rw_ref[...], preferred_element_type=jnp.float32)
    m = jnp.max(logits, axis=1, keepdims=True)
    ex = jnp.exp(logits - m)
    s = jnp.sum(ex, axis=1, keepdims=True)
    probs = ex / s
    gate = jnp.max(probs, axis=1, keepdims=True)
    ids = jax.lax.broadcasted_iota(jnp.int32, probs.shape, 1)
    idx = jnp.min(jnp.where(probs == gate, ids, EXPERTS), axis=1, keepdims=True)
    gate_ref[...] = jnp.broadcast_to(gate, gate_ref.shape)
    idx_ref[...] = jnp.broadcast_to(idx, idx_ref.shape)
    ppsum = jnp.sum(probs, axis=0, keepdims=True)

    @pl.when(i == 0)
    def _():
        psum_ref[...] = ppsum

    @pl.when(i > 0)
    def _():
        psum_ref[...] += ppsum


def _dispatch_body(idx_hbm, gate_hbm, psum_hbm, x_hbm,
                   pos_hbm, xs_hbm, gs_hbm, be_hbm, valid_hbm, aux_hbm,
                   idx_v, gate_v, pos_v, pos8_v, be_v, valid_v,
                   psum_v, aux_v, xrow_v, grow_v, xsem, gsem):
    wid = lax.axis_index("s") * NC + lax.axis_index("c")
    lane = lax.iota(jnp.int32, 16)
    zeros16 = jnp.zeros((16,), jnp.int32)
    one16 = jnp.ones((16,), jnp.int32)

    def mask_i32(m):
        return jnp.where(m, one16, zeros16)

    def full16(val):
        return jnp.full((16,), val, jnp.int32)

    pltpu.sync_copy(idx_hbm, idx_v)
    pltpu.sync_copy(gate_hbm.at[pl.ds(wid * CHUNK, CHUNK)], gate_v)

    def count_body(g, acc):
        v = idx_v[pl.ds(g * 16, 16)]
        return tuple(acc[e] + mask_i32(v == full16(e)) for e in range(EXPERTS))

    zacc = tuple(zeros16 for _ in range(EXPERTS))
    pre_acc = lax.fori_loop(0, wid * NGRP, count_body, zacc)
    tot_acc = lax.fori_loop(wid * NGRP, TOKENS // 16, count_body, pre_acc)
    pre = [jnp.sum(pre_acc[e]) for e in range(EXPERTS)]
    tot = [jnp.sum(tot_acc[e]) for e in range(EXPERTS)]

    tot_vec = zeros16
    for e in range(EXPERTS):
        tot_vec = tot_vec + jnp.where(lane == full16(e), zeros16 + tot[e],
                                      zeros16)
    nbv = (tot_vec + (TM - 1)) >> 9
    pw = nbv << 9
    poff = plsc.cumsum(pw) - pw

    bases = [poff[e] + pre[e] for e in range(EXPERTS)]
    for g in range(NGRP):
        v = idx_v[pl.ds((wid * NGRP + g) * 16, 16)]
        p_vec = zeros16
        for e in range(EXPERTS):
            m = mask_i32(v == full16(e))
            incl = plsc.cumsum(m)
            p_vec = p_vec + m * (bases[e] + incl - 1)
            bases[e] = bases[e] + jnp.sum(m)
        pos_v[pl.ds(g * 16, 16)] = p_vec
        pos8_v[g] = p_vec
    pltpu.sync_copy(pos_v, pos_hbm.at[pl.ds(wid * CHUNK, CHUNK)])

    fz16 = jnp.zeros((16,), jnp.float32)
    for bb in range(2):
        for r in range(16):
            for c in range(GW // 16):
                grow_v[bb, r, pl.ds(c * 16, 16)] = fz16

    xdescs = [None] * NGRP
    gdescs = [None] * NGRP
    for g in range(NGRP):
        if g >= 2:
            xdescs[g - 2].wait()
            gdescs[g - 2].wait()
        pltpu.sync_copy(x_hbm.at[pl.ds(wid * CHUNK + g * 16, 16)],
                        xrow_v.at[g % 2])
        xdescs[g] = pltpu.make_async_copy(xrow_v.at[g % 2],
                                          xs_hbm.at[pos8_v.at[g]], xsem)
        xdescs[g].start()
        gvec = gate_v[pl.ds(g * 16, 16)]
        plsc.store_scatter(grow_v.at[g % 2], [lane, zeros16], gvec)
        gdescs[g] = pltpu.make_async_copy(grow_v.at[g % 2],
                                          gs_hbm.at[pos8_v.at[g]], gsem)
        gdescs[g].start()
    for g in range(NGRP - 2, NGRP):
        xdescs[g].wait()
        gdescs[g].wait()

    @pl.when(wid == 0)
    def _():
        bc = plsc.cumsum(nbv)
        boff = bc - nbv
        nblk_tot = bc[EXPERTS - 1]
        acc = zeros16
        for e in range(EXPERTS):
            acc = acc + mask_i32(lane >= full16(boff[e]))
        be_vec = jnp.minimum(acc - one16, full16(EXPERTS - 1))
        valid_vec = mask_i32(lane < full16(nblk_tot))
        be_v[pl.ds(0, 16)] = be_vec
        be_v[pl.ds(16, 16)] = full16(EXPERTS - 1)
        valid_v[pl.ds(0, 16)] = valid_vec
        valid_v[pl.ds(16, 16)] = zeros16
        pltpu.sync_copy(be_v, be_hbm)
        pltpu.sync_copy(valid_v, valid_hbm)
        pltpu.sync_copy(psum_hbm, psum_v)
        prod = psum_v[...] * tot_vec.astype(jnp.float32)
        aux = jnp.sum(prod) * (LBW * EXPERTS / (float(TOKENS) * float(TOKENS)))
        aux_v[...] = jnp.zeros((16,), jnp.float32) + aux
        pltpu.sync_copy(aux_v, aux_hbm)


def _expert_body(be_ref, valid_ref, x_ref, gs_ref, w1_ref, b1_ref,
                 w2_ref, b2_ref, out_ref):
    s = pl.program_id(0)
    f = pl.program_id(1)

    @pl.when(valid_ref[s] == 1)
    def _():
        sel = (jax.lax.broadcasted_iota(jnp.int32, (GW, 1), 0) == 0)
        gcol = jnp.dot(gs_ref[...], sel.astype(jnp.float32),
                       precision=jax.lax.Precision.HIGHEST,
                       preferred_element_type=jnp.float32)
        h = jnp.dot(x_ref[...], w1_ref[0], preferred_element_type=jnp.float32)
        h = jax.nn.gelu(h + b1_ref[0])
        contrib = jnp.dot(h, w2_ref[0], preferred_element_type=jnp.float32)
        bias_on = jnp.where(f == 0, 1.0, 0.0).astype(jnp.float32)
        contrib = contrib + bias_on * b2_ref[0]
        delta = gcol * contrib

        @pl.when(f == 0)
        def _():
            out_ref[...] = delta

        @pl.when(f > 0)
        def _():
            out_ref[...] += delta


def _combine_body(pos_hbm, outs_hbm, out_hbm, pos_v, pos8_v, row_v, sem):
    wid = lax.axis_index("s") * NC + lax.axis_index("c")
    pltpu.sync_copy(pos_hbm.at[pl.ds(wid * CHUNK, CHUNK)], pos_v)
    for g in range(NGRP):
        pos8_v[g] = pos_v[pl.ds(g * 16, 16)]
    descs = [None] * NGRP
    descs[0] = pltpu.make_async_copy(outs_hbm.at[pos8_v.at[0]],
                                     row_v.at[0], sem)
    descs[0].start()
    for g in range(NGRP):
        descs[g].wait()
        if g + 1 < NGRP:
            descs[g + 1] = pltpu.make_async_copy(
                outs_hbm.at[pos8_v.at[g + 1]], row_v.at[(g + 1) % 2], sem)
            descs[g + 1].start()
        pltpu.sync_copy(row_v.at[g % 2],
                        out_hbm.at[pl.ds(wid * CHUNK + g * 16, 16)])


def _sc_mesh():
    return plsc.VectorSubcoreMesh(core_axis_name="c", subcore_axis_name="s")


def kernel(x, router_w, fc1_w, fc1_b, fc2_w, fc2_b):
    b, s_, h_ = x.shape
    x_flat = x.reshape(-1, h_)
    mesh = _sc_mesh()

    rb = TOKENS
    gate2d, idx2d, psum = pl.pallas_call(
        _router_body,
        grid=(TOKENS // rb,),
        in_specs=[
            pl.BlockSpec((rb, HIDDEN), lambda i: (i, 0)),
            pl.BlockSpec((HIDDEN, EXPERTS), lambda i: (0, 0)),
        ],
        out_specs=[
            pl.BlockSpec((rb, EXPERTS), lambda i: (i, 0)),
            pl.BlockSpec((rb, EXPERTS), lambda i: (i, 0)),
            pl.BlockSpec((1, EXPERTS), lambda i: (0, 0)),
        ],
        out_shape=[
            jax.ShapeDtypeStruct((TOKENS, EXPERTS), jnp.float32),
            jax.ShapeDtypeStruct((TOKENS, EXPERTS), jnp.int32),
            jax.ShapeDtypeStruct((1, EXPERTS), jnp.float32),
        ],
    )(x_flat, router_w)

    idx = idx2d[:, 0]
    gate = gate2d[:, 0]
    psum16 = jnp.pad(psum.reshape(EXPERTS), (0, 8))

    dispatch = pl.kernel(
        _dispatch_body,
        out_type=[
            jax.ShapeDtypeStruct((TOKENS,), jnp.int32),
            jax.ShapeDtypeStruct((TPAD, HIDDEN), jnp.float32),
            jax.ShapeDtypeStruct((TPAD, GW), jnp.float32),
            jax.ShapeDtypeStruct((32,), jnp.int32),
            jax.ShapeDtypeStruct((32,), jnp.int32),
            jax.ShapeDtypeStruct((16,), jnp.float32),
        ],
        mesh=mesh,
        compiler_params=_SC_PARAMS,
        scratch_types=[
            pltpu.VMEM((TOKENS,), jnp.int32),
            pltpu.VMEM((CHUNK,), jnp.float32),
            pltpu.VMEM((CHUNK,), jnp.int32),
            pltpu.VMEM((NGRP, 16), jnp.int32),
            pltpu.VMEM((32,), jnp.int32),
            pltpu.VMEM((32,), jnp.int32),
            pltpu.VMEM((16,), jnp.float32),
            pltpu.VMEM((16,), jnp.float32),
            pltpu.VMEM((2, 16, HIDDEN), jnp.float32),
            pltpu.VMEM((2, 16, GW), jnp.float32),
            pltpu.SemaphoreType.DMA,
            pltpu.SemaphoreType.DMA,
        ],
    )
    posv, xs, gs, bev, validv, auxv = dispatch(idx, gate, psum16, x_flat)

    grid_spec = pltpu.PrefetchScalarGridSpec(
        num_scalar_prefetch=2,
        grid=(NBLKP, NF),
        in_specs=[
            pl.BlockSpec((TM, HIDDEN), lambda st, f, be, vv: (st, 0)),
            pl.BlockSpec((TM, GW), lambda st, f, be, vv: (st, 0)),
            pl.BlockSpec((1, HIDDEN, FK), lambda st, f, be, vv: (be[st], 0, f)),
            pl.BlockSpec((1, 1, FK), lambda st, f, be, vv: (be[st], 0, f)),
            pl.BlockSpec((1, FK, HIDDEN), lambda st, f, be, vv: (be[st], f, 0)),
            pl.BlockSpec((1, 1, HIDDEN), lambda st, f, be, vv: (be[st], 0, 0)),
        ],
        out_specs=pl.BlockSpec((TM, HIDDEN), lambda st, f, be, vv: (st, 0)),
    )
    outs = pl.pallas_call(
        _expert_body,
        grid_spec=grid_spec,
        out_shape=jax.ShapeDtypeStruct((TPAD, HIDDEN), jnp.float32),
    )(bev, validv, xs, gs, fc1_w, fc1_b.reshape(EXPERTS, 1, FFN),
      fc2_w, fc2_b.reshape(EXPERTS, 1, HIDDEN))

    combine = pl.kernel(
        _combine_body,
        out_type=[jax.ShapeDtypeStruct((TOKENS, HIDDEN), jnp.float32)],
        mesh=mesh,
        compiler_params=_SC_PARAMS,
        scratch_types=[
            pltpu.VMEM((CHUNK,), jnp.int32),
            pltpu.VMEM((NGRP, 16), jnp.int32),
            pltpu.VMEM((2, 16, HIDDEN), jnp.float32),
            pltpu.SemaphoreType.DMA,
        ],
    )
    (out,) = combine(posv, outs)

    return out.reshape(b, s_, h_), auxv[0].reshape(())


# speedup vs baseline: 1.1975x; 1.1845x over previous
import jax
import jax.numpy as jnp
from jax import lax
from jax.experimental import pallas as pl
from jax.experimental.pallas import tpu as pltpu
from jax.experimental.pallas import tpu_sc as plsc

HIDDEN = 1024
EXPERTS = 8
FFN = 4096
TOKENS = 4096
LBW = 0.01

TM = 512
NBLKP = TOKENS // TM + EXPERTS - 1
TPAD = NBLKP * TM
FK = 2048
NF = FFN // FK
GW = 128

NC = 2
NS = 16
NW = NC * NS
CHUNK = TOKENS // NW
NGRP = CHUNK // 16

_SC_PARAMS = pltpu.CompilerParams(needs_layout_passes=False)


def _router_body(x_ref, rw_ref, gate_ref, idx_ref, psum_ref):
    i = pl.program_id(0)
    logits = jnp.dot(x_ref[...], rw_ref[...], preferred_element_type=jnp.float32)
    m = jnp.max(logits, axis=1, keepdims=True)
    ex = jnp.exp(logits - m)
    s = jnp.sum(ex, axis=1, keepdims=True)
    probs = ex / s
    gate = jnp.max(probs, axis=1, keepdims=True)
    ids = jax.lax.broadcasted_iota(jnp.int32, probs.shape, 1)
    idx = jnp.min(jnp.where(probs == gate, ids, EXPERTS), axis=1, keepdims=True)
    gate_ref[...] = jnp.broadcast_to(gate, gate_ref.shape)
    idx_ref[...] = jnp.broadcast_to(idx, idx_ref.shape)
    ppsum = jnp.sum(probs, axis=0, keepdims=True)

    @pl.when(i == 0)
    def _():
        psum_ref[...] = ppsum

    @pl.when(i > 0)
    def _():
        psum_ref[...] += ppsum


def _dispatch_body(idx_hbm, gate_hbm, psum_hbm, x_hbm,
                   pos_hbm, xs_hbm, gs_hbm, be_hbm, valid_hbm, aux_hbm,
                   idx_v, gate_v, pos_v, pos8_v, be_v, valid_v,
                   psum_v, aux_v, xrow_v, grow_v, xsem, gsem):
    wid = lax.axis_index("s") * NC + lax.axis_index("c")
    lane = lax.iota(jnp.int32, 16)
    zeros16 = jnp.zeros((16,), jnp.int32)
    one16 = jnp.ones((16,), jnp.int32)

    def mask_i32(m):
        return jnp.where(m, one16, zeros16)

    def full16(val):
        return jnp.full((16,), val, jnp.int32)

    pltpu.sync_copy(idx_hbm, idx_v)
    pltpu.sync_copy(gate_hbm.at[pl.ds(wid * CHUNK, CHUNK)], gate_v)

    def count_body(g, acc):
        v = idx_v[pl.ds(g * 16, 16)]
        return tuple(acc[e] + mask_i32(v == full16(e)) for e in range(EXPERTS))

    zacc = tuple(zeros16 for _ in range(EXPERTS))
    pre_acc = lax.fori_loop(0, wid * NGRP, count_body, zacc)
    tot_acc = lax.fori_loop(wid * NGRP, TOKENS // 16, count_body, pre_acc)
    pre = [jnp.sum(pre_acc[e]) for e in range(EXPERTS)]
    tot = [jnp.sum(tot_acc[e]) for e in range(EXPERTS)]

    tot_vec = zeros16
    for e in range(EXPERTS):
        tot_vec = tot_vec + jnp.where(lane == full16(e), zeros16 + tot[e],
                                      zeros16)
    nbv = (tot_vec + (TM - 1)) >> 9
    pw = nbv << 9
    poff = plsc.cumsum(pw) - pw

    bases = [poff[e] + pre[e] for e in range(EXPERTS)]
    for g in range(NGRP):
        v = idx_v[pl.ds((wid * NGRP + g) * 16, 16)]
        p_vec = zeros16
        for e in range(EXPERTS):
            m = mask_i32(v == full16(e))
            incl = plsc.cumsum(m)
            p_vec = p_vec + m * (bases[e] + incl - 1)
            bases[e] = bases[e] + jnp.sum(m)
        pos_v[pl.ds(g * 16, 16)] = p_vec
        pos8_v[g] = p_vec
    pltpu.sync_copy(pos_v, pos_hbm.at[pl.ds(wid * CHUNK, CHUNK)])

    fz16 = jnp.zeros((16,), jnp.float32)
    for bb in range(2):
        for r in range(16):
            for c in range(GW // 16):
                grow_v[bb, r, pl.ds(c * 16, 16)] = fz16

    xdescs = [None] * NGRP
    gdescs = [None] * NGRP
    for g in range(NGRP):
        if g >= 2:
            xdescs[g - 2].wait()
            gdescs[g - 2].wait()
        pltpu.sync_copy(x_hbm.at[pl.ds(wid * CHUNK + g * 16, 16)],
                        xrow_v.at[g % 2])
        xdescs[g] = pltpu.make_async_copy(xrow_v.at[g % 2],
                                          xs_hbm.at[pos8_v.at[g]], xsem)
        xdescs[g].start()
        gvec = gate_v[pl.ds(g * 16, 16)]
        plsc.store_scatter(grow_v.at[g % 2], [lane, zeros16], gvec)
        gdescs[g] = pltpu.make_async_copy(grow_v.at[g % 2],
                                          gs_hbm.at[pos8_v.at[g]], gsem)
        gdescs[g].start()
    for g in range(NGRP - 2, NGRP):
        xdescs[g].wait()
        gdescs[g].wait()

    @pl.when(wid == 0)
    def _():
        bc = plsc.cumsum(nbv)
        boff = bc - nbv
        nblk_tot = bc[EXPERTS - 1]
        acc = zeros16
        for e in range(EXPERTS):
            acc = acc + mask_i32(lane >= full16(boff[e]))
        be_vec = jnp.minimum(acc - one16, full16(EXPERTS - 1))
        valid_vec = mask_i32(lane < full16(nblk_tot))
        be_v[pl.ds(0, 16)] = be_vec
        be_v[pl.ds(16, 16)] = full16(EXPERTS - 1)
        valid_v[pl.ds(0, 16)] = valid_vec
        valid_v[pl.ds(16, 16)] = zeros16
        pltpu.sync_copy(be_v, be_hbm)
        pltpu.sync_copy(valid_v, valid_hbm)
        pltpu.sync_copy(psum_hbm, psum_v)
        prod = psum_v[...] * tot_vec.astype(jnp.float32)
        aux = jnp.sum(prod) * (LBW * EXPERTS / (float(TOKENS) * float(TOKENS)))
        aux_v[...] = jnp.zeros((16,), jnp.float32) + aux
        pltpu.sync_copy(aux_v, aux_hbm)


def _expert_body(be_ref, valid_ref, x_ref, gs_ref, w1_ref, b1_ref,
                 w2_ref, b2_ref, out_ref):
    s = pl.program_id(0)
    f = pl.program_id(1)

    @pl.when(valid_ref[s] == 1)
    def _():
        sel = (jax.lax.broadcasted_iota(jnp.int32, (GW, 1), 0) == 0)
        gcol = jnp.dot(gs_ref[...], sel.astype(jnp.float32),
                       precision=jax.lax.Precision.HIGHEST,
                       preferred_element_type=jnp.float32)
        h = jnp.dot(x_ref[...], w1_ref[0], preferred_element_type=jnp.float32)
        h = jax.nn.gelu(h + b1_ref[0])
        contrib = jnp.dot(h, w2_ref[0], preferred_element_type=jnp.float32)
        bias_on = jnp.where(f == 0, 1.0, 0.0).astype(jnp.float32)
        contrib = contrib + bias_on * b2_ref[0]
        delta = gcol * contrib

        @pl.when(f == 0)
        def _():
            out_ref[...] = delta

        @pl.when(f > 0)
        def _():
            out_ref[...] += delta


def _combine_body(pos_hbm, outs_hbm, out_hbm, pos_v, pos8_v, row_v, sem):
    wid = lax.axis_index("s") * NC + lax.axis_index("c")
    pltpu.sync_copy(pos_hbm.at[pl.ds(wid * CHUNK, CHUNK)], pos_v)
    for g in range(NGRP):
        pos8_v[g] = pos_v[pl.ds(g * 16, 16)]
    descs = [None] * NGRP
    descs[0] = pltpu.make_async_copy(outs_hbm.at[pos8_v.at[0]],
                                     row_v.at[0], sem)
    descs[0].start()
    for g in range(NGRP):
        descs[g].wait()
        if g + 1 < NGRP:
            descs[g + 1] = pltpu.make_async_copy(
                outs_hbm.at[pos8_v.at[g + 1]], row_v.at[(g + 1) % 2], sem)
            descs[g + 1].start()
        pltpu.sync_copy(row_v.at[g % 2],
                        out_hbm.at[pl.ds(wid * CHUNK + g * 16, 16)])


def _sc_mesh():
    return plsc.VectorSubcoreMesh(core_axis_name="c", subcore_axis_name="s")


def kernel(x, router_w, fc1_w, fc1_b, fc2_w, fc2_b):
    b, s_, h_ = x.shape
    x_flat = x.reshape(-1, h_)
    mesh = _sc_mesh()

    rb = TOKENS
    gate2d, idx2d, psum = pl.pallas_call(
        _router_body,
        grid=(TOKENS // rb,),
        in_specs=[
            pl.BlockSpec((rb, HIDDEN), lambda i: (i, 0)),
            pl.BlockSpec((HIDDEN, EXPERTS), lambda i: (0, 0)),
        ],
        out_specs=[
            pl.BlockSpec((rb, EXPERTS), lambda i: (i, 0)),
            pl.BlockSpec((rb, EXPERTS), lambda i: (i, 0)),
            pl.BlockSpec((1, EXPERTS), lambda i: (0, 0)),
        ],
        out_shape=[
            jax.ShapeDtypeStruct((TOKENS, EXPERTS), jnp.float32),
            jax.ShapeDtypeStruct((TOKENS, EXPERTS), jnp.int32),
            jax.ShapeDtypeStruct((1, EXPERTS), jnp.float32),
        ],
    )(x_flat, router_w)

    idx = idx2d[:, 0]
    gate = gate2d[:, 0]
    psum16 = jnp.pad(psum.reshape(EXPERTS), (0, 8))

    dispatch = pl.kernel(
        _dispatch_body,
        out_type=[
            jax.ShapeDtypeStruct((TOKENS,), jnp.int32),
            jax.ShapeDtypeStruct((TPAD, HIDDEN), jnp.float32),
            jax.ShapeDtypeStruct((TPAD, GW), jnp.float32),
            jax.ShapeDtypeStruct((32,), jnp.int32),
            jax.ShapeDtypeStruct((32,), jnp.int32),
            jax.ShapeDtypeStruct((16,), jnp.float32),
        ],
        mesh=mesh,
        compiler_params=_SC_PARAMS,
        scratch_types=[
            pltpu.VMEM((TOKENS,), jnp.int32),
            pltpu.VMEM((CHUNK,), jnp.float32),
            pltpu.VMEM((CHUNK,), jnp.int32),
            pltpu.VMEM((NGRP, 16), jnp.int32),
            pltpu.VMEM((32,), jnp.int32),
            pltpu.VMEM((32,), jnp.int32),
            pltpu.VMEM((16,), jnp.float32),
            pltpu.VMEM((16,), jnp.float32),
            pltpu.VMEM((2, 16, HIDDEN), jnp.float32),
            pltpu.VMEM((2, 16, GW), jnp.float32),
            pltpu.SemaphoreType.DMA,
            pltpu.SemaphoreType.DMA,
        ],
    )
    posv, xs, gs, bev, validv, auxv = dispatch(idx, gate, psum16, x_flat)

    grid_spec = pltpu.PrefetchScalarGridSpec(
        num_scalar_prefetch=2,
        grid=(NBLKP, NF),
        in_specs=[
            pl.BlockSpec((TM, HIDDEN), lambda st, f, be, vv: (st, 0)),
            pl.BlockSpec((TM, GW), lambda st, f, be, vv: (st, 0)),
            pl.BlockSpec((1, HIDDEN, FK),
                         lambda st, f, be, vv: (be[st], 0,
                                                jnp.where(vv[st] == 1, f, NF - 1))),
            pl.BlockSpec((1, 1, FK),
                         lambda st, f, be, vv: (be[st], 0,
                                                jnp.where(vv[st] == 1, f, NF - 1))),
            pl.BlockSpec((1, FK, HIDDEN),
                         lambda st, f, be, vv: (be[st],
                                                jnp.where(vv[st] == 1, f, NF - 1), 0)),
            pl.BlockSpec((1, 1, HIDDEN), lambda st, f, be, vv: (be[st], 0, 0)),
        ],
        out_specs=pl.BlockSpec((TM, HIDDEN), lambda st, f, be, vv: (st, 0)),
    )
    outs = pl.pallas_call(
        _expert_body,
        grid_spec=grid_spec,
        out_shape=jax.ShapeDtypeStruct((TPAD, HIDDEN), jnp.float32),
    )(bev, validv, xs, gs, fc1_w, fc1_b.reshape(EXPERTS, 1, FFN),
      fc2_w, fc2_b.reshape(EXPERTS, 1, HIDDEN))

    combine = pl.kernel(
        _combine_body,
        out_type=[jax.ShapeDtypeStruct((TOKENS, HIDDEN), jnp.float32)],
        mesh=mesh,
        compiler_params=_SC_PARAMS,
        scratch_types=[
            pltpu.VMEM((CHUNK,), jnp.int32),
            pltpu.VMEM((NGRP, 16), jnp.int32),
            pltpu.VMEM((2, 16, HIDDEN), jnp.float32),
            pltpu.SemaphoreType.DMA,
        ],
    )
    (out,) = combine(posv, outs)

    return out.reshape(b, s_, h_), auxv[0].reshape(())
